# Initial kernel scaffold; baseline (speedup 1.0000x reference)
import jax
import jax.numpy as jnp
from jax.experimental import pallas as pl


def kernel(x, edge_index, W1, b1, W2, b2):
    raise NotImplementedError("write your pallas kernel here")


# speedup vs baseline: 32.6462x; 32.6462x over previous
import functools

import jax
import jax.numpy as jnp
from jax import lax
from jax.experimental import pallas as pl
from jax.experimental.pallas import tpu as pltpu
from jax.experimental.pallas import tpu_sc as plsc

N = 10000
E = 320000
D = 128
H = 16

NC = 2
NS = 16
NW = NC * NS
CHUNK = 128
EPAD = 327680
CPT = EPAD // (NW * CHUNK)
NPAD = 10240
RPT = NPAD // NS
BROW = 1000


def _make_edge_pass(width, gather):
  mesh = plsc.VectorSubcoreMesh(core_axis_name="c", subcore_axis_name="s")
  scratch = [
      pltpu.VMEM((CPT, CHUNK), jnp.int32),
      pltpu.VMEM((CPT, CHUNK), jnp.int32),
      pltpu.VMEM((CHUNK, width), jnp.float32),
      pltpu.VMEM_SHARED((NPAD, width), jnp.float32),
      pltpu.SemaphoreType.DMA,
  ]
  out_t = jax.ShapeDtypeStruct((NC, NPAD, width), jnp.float32)

  @functools.partial(
      pl.kernel, mesh=mesh, out_type=out_t, scratch_types=scratch,
      compiler_params=pltpu.CompilerParams(use_tc_tiling_on_sc=False))
  def k(table_hbm, src_hbm, dst_hbm, zeros_hbm, out_hbm,
        src_v, dst_v, rows_v, acc_s, sem):
    c = lax.axis_index("c")
    s = lax.axis_index("s")
    wid = c * NS + s

    @pl.when(s == 0)
    def _():
      pltpu.sync_copy(zeros_hbm, acc_s)

    if gather:
      pltpu.sync_copy(src_hbm.at[wid], src_v)
    else:
      pltpu.sync_copy(table_hbm, rows_v)
    pltpu.sync_copy(dst_hbm.at[wid], dst_v)
    plsc.subcore_barrier()

    def body(j, carry):
      if gather:
        pltpu.async_copy(table_hbm.at[src_v.at[j]], rows_v, sem).wait()
      pltpu.sync_copy(rows_v, acc_s.at[dst_v.at[j]], add=True)
      return carry

    lax.fori_loop(0, CPT, body, 0)
    plsc.subcore_barrier()
    pltpu.sync_copy(acc_s.at[pl.ds(s * RPT, RPT)],
                    out_hbm.at[c].at[pl.ds(s * RPT, RPT)])

  return k


_deg_pass = _make_edge_pass(1, gather=False)
_edge_pass_h = _make_edge_pass(H, gather=True)
_edge_pass_1 = _make_edge_pass(1, gather=True)


def _k2_body(x_ref, degp_ref, w1_ref, g1_ref, dinv_ref):
  deg = degp_ref[0] + degp_ref[1] + 1.0
  dinv = lax.rsqrt(deg)
  h = jnp.dot(x_ref[...], w1_ref[...], preferred_element_type=jnp.float32)
  g1_ref[...] = h * dinv
  dinv_ref[...] = dinv


def _k2(x, degp, w1):
  grid = N // BROW
  return pl.pallas_call(
      _k2_body,
      grid=(grid,),
      in_specs=[
          pl.BlockSpec((BROW, D), lambda i: (i, 0)),
          pl.BlockSpec((NC, BROW, 1), lambda i: (0, i, 0)),
          pl.BlockSpec((D, H), lambda i: (0, 0)),
      ],
      out_specs=[
          pl.BlockSpec((BROW, H), lambda i: (i, 0)),
          pl.BlockSpec((BROW, 1), lambda i: (i, 0)),
      ],
      out_shape=[
          jax.ShapeDtypeStruct((N, H), jnp.float32),
          jax.ShapeDtypeStruct((N, 1), jnp.float32),
      ],
  )(x, degp, w1)


def _k4_body(accp_ref, g1_ref, dinv_ref, w2t_ref, b1_ref, g2_ref):
  acc = accp_ref[0] + accp_ref[1]
  dinv = dinv_ref[...]
  pre = (acc + g1_ref[...]) * dinv + b1_ref[...]
  out1 = jnp.maximum(pre, 0.0)
  h2 = jnp.sum(out1 * w2t_ref[...], axis=1, keepdims=True)
  g2_ref[...] = h2 * dinv


def _k4(accp, g1, dinv, w2t, b1r):
  grid = N // BROW
  return pl.pallas_call(
      _k4_body,
      grid=(grid,),
      in_specs=[
          pl.BlockSpec((NC, BROW, H), lambda i: (0, i, 0)),
          pl.BlockSpec((BROW, H), lambda i: (i, 0)),
          pl.BlockSpec((BROW, 1), lambda i: (i, 0)),
          pl.BlockSpec((1, H), lambda i: (0, 0)),
          pl.BlockSpec((1, H), lambda i: (0, 0)),
      ],
      out_specs=pl.BlockSpec((BROW, 1), lambda i: (i, 0)),
      out_shape=jax.ShapeDtypeStruct((N, 1), jnp.float32),
  )(accp, g1, dinv, w2t, b1r)


def _k6_body(accp_ref, g2_ref, dinv_ref, b2_ref, out_ref):
  acc = accp_ref[0] + accp_ref[1]
  out_ref[...] = (acc + g2_ref[...]) * dinv_ref[...] + b2_ref[...]


def _k6(accp, g2, dinv, b2r):
  grid = N // BROW
  return pl.pallas_call(
      _k6_body,
      grid=(grid,),
      in_specs=[
          pl.BlockSpec((NC, BROW, 1), lambda i: (0, i, 0)),
          pl.BlockSpec((BROW, 1), lambda i: (i, 0)),
          pl.BlockSpec((BROW, 1), lambda i: (i, 0)),
          pl.BlockSpec((1, 1), lambda i: (0, 0)),
      ],
      out_specs=pl.BlockSpec((BROW, 1), lambda i: (i, 0)),
      out_shape=jax.ShapeDtypeStruct((N, 1), jnp.float32),
  )(accp, g2, dinv, b2r)


def kernel(x, edge_index, W1, b1, W2, b2):
  x = x.astype(jnp.float32)
  src = edge_index[0].astype(jnp.int32)
  dst = edge_index[1].astype(jnp.int32)
  padlen = EPAD - E
  fill = jnp.full((padlen,), N, jnp.int32)
  src3 = jnp.concatenate([src, fill]).reshape(NW, CPT, CHUNK)
  dst3 = jnp.concatenate([dst, fill]).reshape(NW, CPT, CHUNK)

  zeros1 = jnp.zeros((NPAD, 1), jnp.float32)
  zerosH = jnp.zeros((NPAD, H), jnp.float32)
  ones_chunk = jnp.ones((CHUNK, 1), jnp.float32)

  degp = _deg_pass(ones_chunk, src3, dst3, zeros1)
  g1, dinv = _k2(x, degp, W1.astype(jnp.float32))

  g1p = jnp.pad(g1, ((0, NPAD - N), (0, 0)))
  acc1p = _edge_pass_h(g1p, src3, dst3, zerosH)

  w2t = W2.astype(jnp.float32).reshape(1, H)
  b1r = b1.astype(jnp.float32).reshape(1, H)
  g2 = _k4(acc1p, g1, dinv, w2t, b1r)

  g2p = jnp.pad(g2, ((0, NPAD - N), (0, 0)))
  acc2p = _edge_pass_1(g2p, src3, dst3, zeros1)

  b2r = b2.astype(jnp.float32).reshape(1, 1)
  return _k6(acc2p, g2, dinv, b2r)


# speedup vs baseline: 42.5499x; 1.3034x over previous
import functools

import jax
import jax.numpy as jnp
from jax import lax
from jax.experimental import pallas as pl
from jax.experimental.pallas import tpu as pltpu
from jax.experimental.pallas import tpu_sc as plsc

N = 10000
E = 320000
D = 128
H = 16

NC = 2
NS = 16
NW = NC * NS
CHUNK = 128
EPAD = 327680
CPT = EPAD // (NW * CHUNK)
NPAD = 10240
RPT = NPAD // NS
BROW = 1000


NBUF = 8
HB = NBUF // 2
NGRP = CPT // HB


def _make_edge_pass(width, gather):
  mesh = plsc.VectorSubcoreMesh(core_axis_name="c", subcore_axis_name="s")
  scratch = [
      pltpu.VMEM((CPT, CHUNK), jnp.int32),
      pltpu.VMEM((CPT, CHUNK), jnp.int32),
      pltpu.VMEM((NBUF, CHUNK, width), jnp.float32),
      pltpu.VMEM_SHARED((NPAD, width), jnp.float32),
      pltpu.SemaphoreType.DMA((NBUF,)),
      pltpu.SemaphoreType.DMA((NBUF,)),
  ]
  out_t = jax.ShapeDtypeStruct((NC, NPAD, width), jnp.float32)

  @functools.partial(
      pl.kernel, mesh=mesh, out_type=out_t, scratch_types=scratch,
      compiler_params=pltpu.CompilerParams(use_tc_tiling_on_sc=False))
  def k(table_hbm, src_hbm, dst_hbm, zeros_hbm, out_hbm,
        src_v, dst_v, rows_v, acc_s, gsem, ssem):
    c = lax.axis_index("c")
    s = lax.axis_index("s")
    wid = c * NS + s

    @pl.when(s == 0)
    def _():
      pltpu.sync_copy(zeros_hbm, acc_s)

    if gather:
      pltpu.sync_copy(src_hbm.at[wid], src_v)
    pltpu.sync_copy(dst_hbm.at[wid], dst_v)
    plsc.subcore_barrier()

    def fire_scatter(j, b):
      pltpu.async_copy(rows_v.at[b], acc_s.at[dst_v.at[j]], ssem.at[b],
                       add=True)

    def wait_scatter(j, b):
      pltpu.make_async_copy(rows_v.at[b], acc_s.at[dst_v.at[j]],
                            ssem.at[b]).wait()

    if gather:
      def fire_gather(j, b):
        pltpu.async_copy(table_hbm.at[src_v.at[j]], rows_v.at[b],
                         gsem.at[b])

      def wait_gather(j, b):
        pltpu.make_async_copy(table_hbm.at[src_v.at[j]], rows_v.at[b],
                              gsem.at[b]).wait()

      for b in range(HB):
        fire_gather(b, b)
      for b in range(HB):
        fire_gather(HB + b, HB + b)

      def body(g, carry):
        base = (g % 2) * HB
        for b in range(HB):
          j = g * HB + b
          wait_gather(j, base + b)
          fire_scatter(j, base + b)
        for b in range(HB):
          j = g * HB + b
          wait_scatter(j, base + b)

          @pl.when(g + 2 < NGRP)
          def _(b=b, j=j, base=base):
            fire_gather(j + 2 * HB, base + b)
        return carry

      lax.fori_loop(0, NGRP, body, 0)
    else:
      pltpu.sync_copy(table_hbm, rows_v.at[0])
      for b in range(NBUF):
        fire_scatter(b, 0)

      def body(g, carry):
        for b in range(NBUF):
          j = g * NBUF + b
          wait_scatter(j, 0)
          fire_scatter(j + NBUF, 0)
        return carry

      lax.fori_loop(0, CPT // NBUF - 1, body, 0)
      for b in range(NBUF):
        wait_scatter(CPT - NBUF + b, 0)

    plsc.subcore_barrier()
    pltpu.sync_copy(acc_s.at[pl.ds(s * RPT, RPT)],
                    out_hbm.at[c].at[pl.ds(s * RPT, RPT)])

  return k


_deg_pass = _make_edge_pass(1, gather=False)
_edge_pass_h = _make_edge_pass(H, gather=True)
_edge_pass_1 = _make_edge_pass(1, gather=True)


def _k2_body(x_ref, degp_ref, w1_ref, g1_ref, dinv_ref):
  deg = degp_ref[0] + degp_ref[1] + 1.0
  dinv = lax.rsqrt(deg)
  h = jnp.dot(x_ref[...], w1_ref[...], preferred_element_type=jnp.float32)
  g1_ref[...] = h * dinv
  dinv_ref[...] = dinv


def _k2(x, degp, w1):
  grid = N // BROW
  return pl.pallas_call(
      _k2_body,
      grid=(grid,),
      in_specs=[
          pl.BlockSpec((BROW, D), lambda i: (i, 0)),
          pl.BlockSpec((NC, BROW, 1), lambda i: (0, i, 0)),
          pl.BlockSpec((D, H), lambda i: (0, 0)),
      ],
      out_specs=[
          pl.BlockSpec((BROW, H), lambda i: (i, 0)),
          pl.BlockSpec((BROW, 1), lambda i: (i, 0)),
      ],
      out_shape=[
          jax.ShapeDtypeStruct((N, H), jnp.float32),
          jax.ShapeDtypeStruct((N, 1), jnp.float32),
      ],
  )(x, degp, w1)


def _k4_body(accp_ref, g1_ref, dinv_ref, w2t_ref, b1_ref, g2_ref):
  acc = accp_ref[0] + accp_ref[1]
  dinv = dinv_ref[...]
  pre = (acc + g1_ref[...]) * dinv + b1_ref[...]
  out1 = jnp.maximum(pre, 0.0)
  h2 = jnp.sum(out1 * w2t_ref[...], axis=1, keepdims=True)
  g2_ref[...] = h2 * dinv


def _k4(accp, g1, dinv, w2t, b1r):
  grid = N // BROW
  return pl.pallas_call(
      _k4_body,
      grid=(grid,),
      in_specs=[
          pl.BlockSpec((NC, BROW, H), lambda i: (0, i, 0)),
          pl.BlockSpec((BROW, H), lambda i: (i, 0)),
          pl.BlockSpec((BROW, 1), lambda i: (i, 0)),
          pl.BlockSpec((1, H), lambda i: (0, 0)),
          pl.BlockSpec((1, H), lambda i: (0, 0)),
      ],
      out_specs=pl.BlockSpec((BROW, 1), lambda i: (i, 0)),
      out_shape=jax.ShapeDtypeStruct((N, 1), jnp.float32),
  )(accp, g1, dinv, w2t, b1r)


def _k6_body(accp_ref, g2_ref, dinv_ref, b2_ref, out_ref):
  acc = accp_ref[0] + accp_ref[1]
  out_ref[...] = (acc + g2_ref[...]) * dinv_ref[...] + b2_ref[...]


def _k6(accp, g2, dinv, b2r):
  grid = N // BROW
  return pl.pallas_call(
      _k6_body,
      grid=(grid,),
      in_specs=[
          pl.BlockSpec((NC, BROW, 1), lambda i: (0, i, 0)),
          pl.BlockSpec((BROW, 1), lambda i: (i, 0)),
          pl.BlockSpec((BROW, 1), lambda i: (i, 0)),
          pl.BlockSpec((1, 1), lambda i: (0, 0)),
      ],
      out_specs=pl.BlockSpec((BROW, 1), lambda i: (i, 0)),
      out_shape=jax.ShapeDtypeStruct((N, 1), jnp.float32),
  )(accp, g2, dinv, b2r)


def kernel(x, edge_index, W1, b1, W2, b2):
  x = x.astype(jnp.float32)
  src = edge_index[0].astype(jnp.int32)
  dst = edge_index[1].astype(jnp.int32)
  padlen = EPAD - E
  fill = jnp.full((padlen,), N, jnp.int32)
  src3 = jnp.concatenate([src, fill]).reshape(NW, CPT, CHUNK)
  dst3 = jnp.concatenate([dst, fill]).reshape(NW, CPT, CHUNK)

  zeros1 = jnp.zeros((NPAD, 1), jnp.float32)
  zerosH = jnp.zeros((NPAD, H), jnp.float32)
  ones_chunk = jnp.ones((CHUNK, 1), jnp.float32)

  degp = _deg_pass(ones_chunk, src3, dst3, zeros1)
  g1, dinv = _k2(x, degp, W1.astype(jnp.float32))

  g1p = jnp.pad(g1, ((0, NPAD - N), (0, 0)))
  acc1p = _edge_pass_h(g1p, src3, dst3, zerosH)

  w2t = W2.astype(jnp.float32).reshape(1, H)
  b1r = b1.astype(jnp.float32).reshape(1, H)
  g2 = _k4(acc1p, g1, dinv, w2t, b1r)

  g2p = jnp.pad(g2, ((0, NPAD - N), (0, 0)))
  acc2p = _edge_pass_1(g2p, src3, dst3, zeros1)

  b2r = b2.astype(jnp.float32).reshape(1, 1)
  return _k6(acc2p, g2, dinv, b2r)
